# Initial kernel scaffold; baseline (speedup 1.0000x reference)
import jax
import jax.numpy as jnp
from jax.experimental import pallas as pl


def kernel(output, target, one_hot):
    raise NotImplementedError("write your pallas kernel here")


# speedup vs baseline: 2.3941x; 2.3941x over previous
import functools

import jax
import jax.numpy as jnp
from jax.experimental import pallas as pl
from jax.experimental.pallas import tpu as pltpu

LS = 0.1
V = 100000
CONF = 1.0 - LS
SMOOTH = LS / (V - 2)


def _loss_kernel(target_ref, out_blk_ref, o_ref, acc_ref, *, vb, nblk):
    j = pl.program_id(0)
    blk = out_blk_ref[...]
    b = blk.shape[0]
    col = jax.lax.broadcasted_iota(jnp.int32, (b, vb), 1) + j * vb
    tgt = target_ref[...]
    blk = jnp.where(col < V, blk, 0.0)
    w = jnp.where(col == tgt, CONF, jnp.where(col == 0, 0.0, SMOOTH))
    partial = jnp.sum(blk * w)

    @pl.when(j == 0)
    def _init():
        n0 = jnp.sum(jnp.where(tgt == 0, 1.0, 0.0))
        s32 = jnp.float32(SMOOTH)
        const = b * (jnp.float32(CONF) * jnp.log(jnp.float32(CONF))
                     + (V - 2) * s32 * jnp.log(s32))
        acc_ref[0] = const + n0 * s32 * jnp.log(s32)

    acc_ref[0] = acc_ref[0] - partial

    @pl.when(j == nblk - 1)
    def _done():
        o_ref[0, 0] = acc_ref[0]


def kernel(output, target, one_hot):
    del one_hot
    b, v = output.shape
    vb = 2048
    nblk = pl.cdiv(v, vb)
    tgt2d = target.reshape(b, 1)
    out = pl.pallas_call(
        functools.partial(_loss_kernel, vb=vb, nblk=nblk),
        grid=(nblk,),
        in_specs=[
            pl.BlockSpec((b, 1), lambda j: (0, 0)),
            pl.BlockSpec((b, vb), lambda j: (0, j)),
        ],
        out_specs=pl.BlockSpec(memory_space=pltpu.SMEM),
        out_shape=jax.ShapeDtypeStruct((1, 1), jnp.float32),
        scratch_shapes=[pltpu.SMEM((1,), jnp.float32)],
    )(tgt2d, output)
    return out[0, 0]
